# baseline (speedup 1.0000x reference)
import functools

import jax
import jax.numpy as jnp
from jax import lax
from jax.experimental import pallas as pl
from jax.experimental.pallas import tpu as pltpu
from jax.experimental.pallas import tpu_sc as plsc

Q = 1024
D = 512
K = 100000
BK = 2048
NB = (K + BK - 1) // BK
TOPS = 10
SN = 64
DO = 256
TEMP = 0.07
NEG = -3.0e38
BIG = 1 << 30

_DOT = dict(preferred_element_type=jnp.float32, precision=lax.Precision.HIGHEST)


def _k1_body(scene_ref, g_ref, t_ref):
    i = pl.program_id(0)

    @pl.when(i == 0)
    def _init():
        t_ref[...] = jnp.full((Q, TOPS * 128), NEG, jnp.float32)

    scene = scene_ref[...]
    g = g_ref[...]
    nsub = 4
    csub = BK // nsub
    ssub = [
        lax.dot_general(scene, g[u * csub:(u + 1) * csub, :],
                        (((1,), (1,)), ((), ())), **_DOT)
        for u in range(nsub)
    ]
    first_bad_chunk = (K - (NB - 1) * BK) // 128

    t = [t_ref[:, 128 * j:128 * (j + 1)] for j in range(TOPS)]
    for c in range(BK // 128):
        v = ssub[c // (csub // 128)][:, 128 * (c % (csub // 128)):
                                     128 * (c % (csub // 128)) + 128]
        if c >= first_bad_chunk:
            col = i * BK + 128 * c + lax.broadcasted_iota(
                jnp.int32, (Q, 128), 1)
            v = jnp.where(col < K, v, NEG)
        for j in range(TOPS):
            hi = jnp.maximum(t[j], v)
            v = jnp.minimum(t[j], v)
            t[j] = hi
    for j in range(TOPS):
        t_ref[:, 128 * j:128 * (j + 1)] = t[j]


def _k1(scene_x, graph_x):
    return pl.pallas_call(
        _k1_body,
        grid=(NB,),
        in_specs=[
            pl.BlockSpec((Q, D), lambda i: (0, 0)),
            pl.BlockSpec((BK, D), lambda i: (i, 0)),
        ],
        out_specs=pl.BlockSpec((Q, TOPS * 128), lambda i: (0, 0)),
        out_shape=jax.ShapeDtypeStruct((Q, TOPS * 128), jnp.float32),
        compiler_params=pltpu.CompilerParams(
            dimension_semantics=("arbitrary",)),
    )(scene_x, graph_x)


def _k1b_body(t_ref, scene_ref, md_ref, y_ref):
    tt = t_ref[...]
    ids = lax.broadcasted_iota(jnp.int32, (Q, TOPS * 128), 1)
    acc = jnp.zeros((Q, 1), jnp.float32)
    rowmax = jnp.max(tt, axis=1, keepdims=True)
    cur = tt
    for it in range(TOPS):
        m = jnp.max(cur, axis=1, keepdims=True)
        sel = jnp.where(cur == m, ids, BIG)
        am = jnp.min(sel, axis=1, keepdims=True)
        cur = jnp.where(ids == am, NEG, cur)
        acc = acc + m
    md_ref[...] = acc / TOPS
    gm = jnp.max(rowmax)
    rid = lax.broadcasted_iota(jnp.int32, (Q, 1), 0)
    qid = jnp.min(jnp.where(rowmax == gm, rid, BIG))
    pick = lax.broadcasted_iota(jnp.int32, (Q, D), 0) == qid
    y_ref[...] = jnp.sum(jnp.where(pick, scene_ref[...], 0.0), axis=0,
                         keepdims=True)


def _k1b(t, scene_x):
    return pl.pallas_call(
        _k1b_body,
        in_specs=[
            pl.BlockSpec((Q, TOPS * 128), lambda: (0, 0)),
            pl.BlockSpec((Q, D), lambda: (0, 0)),
        ],
        out_specs=[
            pl.BlockSpec((Q, 1), lambda: (0, 0)),
            pl.BlockSpec((1, D), lambda: (0, 0)),
        ],
        out_shape=[
            jax.ShapeDtypeStruct((Q, 1), jnp.float32),
            jax.ShapeDtypeStruct((1, D), jnp.float32),
        ],
    )(t, scene_x)


BK2 = 8192
NB2 = (K + BK2 - 1) // BK2


def _k2_body(y_ref, g_ref, cos_ref):
    i = pl.program_id(0)
    y = y_ref[...]
    ynorm = jnp.sqrt(jnp.sum(y * y))
    g = g_ref[...]
    z = lax.dot_general(g, y, (((1,), (1,)), ((), ())), **_DOT)
    nsq = jnp.sum(g * g, axis=1, keepdims=True)
    cos = z / (jnp.sqrt(nsq) * ynorm + 1e-8)
    row = i * BK2 + lax.broadcasted_iota(jnp.int32, (BK2, 1), 0)
    cos_ref[...] = jnp.where(row < K, cos, NEG)


def _k2(y, graph_x):
    return pl.pallas_call(
        _k2_body,
        grid=(NB2,),
        in_specs=[
            pl.BlockSpec((1, D), lambda i: (0, 0)),
            pl.BlockSpec((BK2, D), lambda i: (i, 0)),
        ],
        out_specs=pl.BlockSpec((BK2, 1), lambda i: (i, 0)),
        out_shape=jax.ShapeDtypeStruct((NB2 * BK2, 1), jnp.float32),
        compiler_params=pltpu.CompilerParams(
            dimension_semantics=("arbitrary",)),
    )(y, graph_x)


def _k3_body(cos_ref, idx_ref):
    cur = cos_ref[...]
    nbr, nc = cur.shape
    ids = (lax.broadcasted_iota(jnp.int32, (nbr, nc), 0) * nc
           + lax.broadcasted_iota(jnp.int32, (nbr, nc), 1))
    out = jnp.full((SN, 1), 0, jnp.int32)
    oid = lax.broadcasted_iota(jnp.int32, (SN, 1), 0)
    for it in range(SN):
        m = jnp.max(cur)
        g = jnp.min(jnp.where(cur == m, ids, BIG))
        out = jnp.where(oid == it, g, out)
        cur = jnp.where(ids == g, NEG, cur)
    idx_ref[...] = out


def _k3(cos):
    r3, c3 = NB2 * BK2 // 2048, 2048
    return pl.pallas_call(
        _k3_body,
        in_specs=[pl.BlockSpec((r3, c3), lambda: (0, 0))],
        out_specs=pl.BlockSpec((SN, 1), lambda: (0, 0)),
        out_shape=jax.ShapeDtypeStruct((SN, 1), jnp.int32),
    )(cos.reshape(r3, c3))


def _sc_gather_build():
    mesh = plsc.VectorSubcoreMesh(core_axis_name="c", subcore_axis_name="s")
    nw = 8
    rows_per_w = SN // nw

    @functools.partial(
        pl.kernel,
        mesh=mesh,
        out_type=jax.ShapeDtypeStruct((SN, D), jnp.float32),
        scratch_types=[
            pltpu.VMEM((rows_per_w,), jnp.int32),
            pltpu.VMEM((rows_per_w, D), jnp.float32),
            pltpu.SemaphoreType.DMA,
        ],
    )
    def gather(idx_hbm, table_hbm, out_hbm, idx_v, rows_v, sem):
        wid = lax.axis_index("s") * 2 + lax.axis_index("c")

        @pl.when(wid < nw)
        def _():
            base = wid * rows_per_w
            pltpu.sync_copy(idx_hbm.at[pl.ds(base, rows_per_w)], idx_v)
            pltpu.async_copy(table_hbm.at[idx_v], rows_v, sem).wait()
            pltpu.sync_copy(rows_v, out_hbm.at[pl.ds(base, rows_per_w)])

    return gather


_sc_gather_cache = []


def _sc_gather(cidx, graph_x):
    if not _sc_gather_cache:
        _sc_gather_cache.append(_sc_gather_build())
    return _sc_gather_cache[0](cidx, graph_x)


def _k4_body(scene_ref, sub_ref, w_ref, b_ref, tmv_ref, tsub_ref, semb_ref):
    scene = scene_ref[...]
    w = w_ref[...]
    b = b_ref[...]
    mean = jnp.mean(scene, axis=0, keepdims=True)
    semb = jnp.maximum(
        lax.dot_general(mean, w, (((1,), (0,)), ((), ())), **_DOT) + b, 0.0)
    sub = sub_ref[...]
    sp = jnp.maximum(
        lax.dot_general(sub, w, (((1,), (0,)), ((), ())), **_DOT) + b, 0.0)
    logits = lax.dot_general(sp, semb, (((1,), (1,)), ((), ())), **_DOT) / TEMP
    match = jax.nn.sigmoid(logits)
    ids = lax.broadcasted_iota(jnp.int32, (SN, 1), 0)
    oid = lax.broadcasted_iota(jnp.int32, (TOPS, 1), 0)
    rsel = lax.broadcasted_iota(jnp.int32, (TOPS, DO), 0)
    tmv = jnp.zeros((TOPS, 1), jnp.float32)
    tsub = jnp.zeros((TOPS, DO), jnp.float32)
    cur = match
    for it in range(TOPS):
        m = jnp.max(cur)
        am = jnp.min(jnp.where(cur == m, ids, BIG))
        tmv = jnp.where(oid == it, m, tmv)
        row = jnp.sum(jnp.where(ids == am, sp, 0.0), axis=0, keepdims=True)
        tsub = jnp.where(rsel == it, row, tsub)
        cur = jnp.where(ids == am, NEG, cur)
    tmv_ref[...] = tmv
    tsub_ref[...] = tsub
    semb_ref[...] = semb


def _k4(scene_x, sub_embs, w, b2):
    return pl.pallas_call(
        _k4_body,
        in_specs=[
            pl.BlockSpec((Q, D), lambda: (0, 0)),
            pl.BlockSpec((SN, D), lambda: (0, 0)),
            pl.BlockSpec((D, DO), lambda: (0, 0)),
            pl.BlockSpec((1, DO), lambda: (0, 0)),
        ],
        out_specs=[
            pl.BlockSpec((TOPS, 1), lambda: (0, 0)),
            pl.BlockSpec((TOPS, DO), lambda: (0, 0)),
            pl.BlockSpec((1, DO), lambda: (0, 0)),
        ],
        out_shape=[
            jax.ShapeDtypeStruct((TOPS, 1), jnp.float32),
            jax.ShapeDtypeStruct((TOPS, DO), jnp.float32),
            jax.ShapeDtypeStruct((1, DO), jnp.float32),
        ],
    )(scene_x, sub_embs, w, b2)


def kernel(scene_x, graph_x, W, b):
    t = _k1(scene_x, graph_x)
    md, y = _k1b(t, scene_x)
    cos = _k2(y, graph_x)
    cidx = _k3(cos)[:, 0]
    sub_embs = _sc_gather(cidx, graph_x)
    tmv, tsub, semb = _k4(scene_x, sub_embs, W, jnp.reshape(b, (1, DO)))
    return tmv[:, 0], tsub, semb[0], md[:, 0]


# speedup vs baseline: 1.7302x; 1.7302x over previous
import functools

import jax
import jax.numpy as jnp
from jax import lax
from jax.experimental import pallas as pl
from jax.experimental.pallas import tpu as pltpu
from jax.experimental.pallas import tpu_sc as plsc

Q = 1024
D = 512
K = 100000
BK = 2048
NB = (K + BK - 1) // BK
TOPS = 10
SN = 64
DO = 256
TEMP = 0.07
NEG = -3.0e38
BIG = 1 << 30

_DOT = dict(preferred_element_type=jnp.float32, precision=lax.Precision.DEFAULT)


def _k1_body(scene_ref, g_ref, t_ref):
    i = pl.program_id(0)

    @pl.when(i == 0)
    def _init():
        t_ref[...] = jnp.full((Q, TOPS * 128), NEG, jnp.float32)

    scene = scene_ref[...]
    g = g_ref[...]
    nsub = 4
    csub = BK // nsub
    ssub = [
        lax.dot_general(scene, g[u * csub:(u + 1) * csub, :],
                        (((1,), (1,)), ((), ())), **_DOT)
        for u in range(nsub)
    ]
    first_bad_chunk = (K - (NB - 1) * BK) // 128

    t = [t_ref[:, 128 * j:128 * (j + 1)] for j in range(TOPS)]
    for c in range(BK // 128):
        v = ssub[c // (csub // 128)][:, 128 * (c % (csub // 128)):
                                     128 * (c % (csub // 128)) + 128]
        if c >= first_bad_chunk:
            col = i * BK + 128 * c + lax.broadcasted_iota(
                jnp.int32, (Q, 128), 1)
            v = jnp.where(col < K, v, NEG)
        for j in range(TOPS):
            hi = jnp.maximum(t[j], v)
            v = jnp.minimum(t[j], v)
            t[j] = hi
    for j in range(TOPS):
        t_ref[:, 128 * j:128 * (j + 1)] = t[j]


def _k1(scene_x, graph_x):
    return pl.pallas_call(
        _k1_body,
        grid=(NB,),
        in_specs=[
            pl.BlockSpec((Q, D), lambda i: (0, 0)),
            pl.BlockSpec((BK, D), lambda i: (i, 0)),
        ],
        out_specs=pl.BlockSpec((Q, TOPS * 128), lambda i: (0, 0)),
        out_shape=jax.ShapeDtypeStruct((Q, TOPS * 128), jnp.float32),
        compiler_params=pltpu.CompilerParams(
            dimension_semantics=("arbitrary",)),
    )(scene_x, graph_x)


def _k1b_body(t_ref, scene_ref, md_ref, y_ref):
    tt = t_ref[...]
    ids = lax.broadcasted_iota(jnp.int32, (Q, TOPS * 128), 1)
    acc = jnp.zeros((Q, 1), jnp.float32)
    rowmax = jnp.max(tt, axis=1, keepdims=True)
    cur = tt
    for it in range(TOPS):
        m = jnp.max(cur, axis=1, keepdims=True)
        sel = jnp.where(cur == m, ids, BIG)
        am = jnp.min(sel, axis=1, keepdims=True)
        cur = jnp.where(ids == am, NEG, cur)
        acc = acc + m
    md_ref[...] = acc / TOPS
    gm = jnp.max(rowmax)
    rid = lax.broadcasted_iota(jnp.int32, (Q, 1), 0)
    qid = jnp.min(jnp.where(rowmax == gm, rid, BIG))
    pick = lax.broadcasted_iota(jnp.int32, (Q, D), 0) == qid
    y_ref[...] = jnp.sum(jnp.where(pick, scene_ref[...], 0.0), axis=0,
                         keepdims=True)


def _k1b(t, scene_x):
    return pl.pallas_call(
        _k1b_body,
        in_specs=[
            pl.BlockSpec((Q, TOPS * 128), lambda: (0, 0)),
            pl.BlockSpec((Q, D), lambda: (0, 0)),
        ],
        out_specs=[
            pl.BlockSpec((Q, 1), lambda: (0, 0)),
            pl.BlockSpec((1, D), lambda: (0, 0)),
        ],
        out_shape=[
            jax.ShapeDtypeStruct((Q, 1), jnp.float32),
            jax.ShapeDtypeStruct((1, D), jnp.float32),
        ],
    )(t, scene_x)


BK2 = 8192
NB2 = (K + BK2 - 1) // BK2


def _k2_body(y_ref, g_ref, cos_ref):
    i = pl.program_id(0)
    y = y_ref[...]
    ynorm = jnp.sqrt(jnp.sum(y * y))
    g = g_ref[...]
    z = lax.dot_general(g, y, (((1,), (1,)), ((), ())), **_DOT)
    nsq = jnp.sum(g * g, axis=1, keepdims=True)
    cos = z / (jnp.sqrt(nsq) * ynorm + 1e-8)
    row = i * BK2 + lax.broadcasted_iota(jnp.int32, (BK2, 1), 0)
    cos_ref[...] = jnp.where(row < K, cos, NEG)


def _k2(y, graph_x):
    return pl.pallas_call(
        _k2_body,
        grid=(NB2,),
        in_specs=[
            pl.BlockSpec((1, D), lambda i: (0, 0)),
            pl.BlockSpec((BK2, D), lambda i: (i, 0)),
        ],
        out_specs=pl.BlockSpec((BK2, 1), lambda i: (i, 0)),
        out_shape=jax.ShapeDtypeStruct((NB2 * BK2, 1), jnp.float32),
        compiler_params=pltpu.CompilerParams(
            dimension_semantics=("arbitrary",)),
    )(y, graph_x)


def _k3_body(cos_ref, idx_ref):
    cur = cos_ref[...]
    nbr, nc = cur.shape
    ids = (lax.broadcasted_iota(jnp.int32, (nbr, nc), 0) * nc
           + lax.broadcasted_iota(jnp.int32, (nbr, nc), 1))
    out = jnp.full((SN, 1), 0, jnp.int32)
    oid = lax.broadcasted_iota(jnp.int32, (SN, 1), 0)
    for it in range(SN):
        m = jnp.max(cur)
        g = jnp.min(jnp.where(cur == m, ids, BIG))
        out = jnp.where(oid == it, g, out)
        cur = jnp.where(ids == g, NEG, cur)
    idx_ref[...] = out


def _k3(cos):
    r3, c3 = NB2 * BK2 // 2048, 2048
    return pl.pallas_call(
        _k3_body,
        in_specs=[pl.BlockSpec((r3, c3), lambda: (0, 0))],
        out_specs=pl.BlockSpec((SN, 1), lambda: (0, 0)),
        out_shape=jax.ShapeDtypeStruct((SN, 1), jnp.int32),
    )(cos.reshape(r3, c3))


def _sc_gather_build():
    mesh = plsc.VectorSubcoreMesh(core_axis_name="c", subcore_axis_name="s")
    nw = 8
    rows_per_w = SN // nw

    @functools.partial(
        pl.kernel,
        mesh=mesh,
        out_type=jax.ShapeDtypeStruct((SN, D), jnp.float32),
        scratch_types=[
            pltpu.VMEM((rows_per_w,), jnp.int32),
            pltpu.VMEM((rows_per_w, D), jnp.float32),
            pltpu.SemaphoreType.DMA,
        ],
    )
    def gather(idx_hbm, table_hbm, out_hbm, idx_v, rows_v, sem):
        wid = lax.axis_index("s") * 2 + lax.axis_index("c")

        @pl.when(wid < nw)
        def _():
            base = wid * rows_per_w
            pltpu.sync_copy(idx_hbm.at[pl.ds(base, rows_per_w)], idx_v)
            pltpu.async_copy(table_hbm.at[idx_v], rows_v, sem).wait()
            pltpu.sync_copy(rows_v, out_hbm.at[pl.ds(base, rows_per_w)])

    return gather


_sc_gather_cache = []


def _sc_gather(cidx, graph_x):
    if not _sc_gather_cache:
        _sc_gather_cache.append(_sc_gather_build())
    return _sc_gather_cache[0](cidx, graph_x)


def _k4_body(scene_ref, sub_ref, w_ref, b_ref, tmv_ref, tsub_ref, semb_ref):
    scene = scene_ref[...]
    w = w_ref[...]
    b = b_ref[...]
    mean = jnp.mean(scene, axis=0, keepdims=True)
    semb = jnp.maximum(
        lax.dot_general(mean, w, (((1,), (0,)), ((), ())), **_DOT) + b, 0.0)
    sub = sub_ref[...]
    sp = jnp.maximum(
        lax.dot_general(sub, w, (((1,), (0,)), ((), ())), **_DOT) + b, 0.0)
    logits = lax.dot_general(sp, semb, (((1,), (1,)), ((), ())), **_DOT) / TEMP
    match = jax.nn.sigmoid(logits)
    ids = lax.broadcasted_iota(jnp.int32, (SN, 1), 0)
    oid = lax.broadcasted_iota(jnp.int32, (TOPS, 1), 0)
    rsel = lax.broadcasted_iota(jnp.int32, (TOPS, DO), 0)
    tmv = jnp.zeros((TOPS, 1), jnp.float32)
    tsub = jnp.zeros((TOPS, DO), jnp.float32)
    cur = match
    for it in range(TOPS):
        m = jnp.max(cur)
        am = jnp.min(jnp.where(cur == m, ids, BIG))
        tmv = jnp.where(oid == it, m, tmv)
        row = jnp.sum(jnp.where(ids == am, sp, 0.0), axis=0, keepdims=True)
        tsub = jnp.where(rsel == it, row, tsub)
        cur = jnp.where(ids == am, NEG, cur)
    tmv_ref[...] = tmv
    tsub_ref[...] = tsub
    semb_ref[...] = semb


def _k4(scene_x, sub_embs, w, b2):
    return pl.pallas_call(
        _k4_body,
        in_specs=[
            pl.BlockSpec((Q, D), lambda: (0, 0)),
            pl.BlockSpec((SN, D), lambda: (0, 0)),
            pl.BlockSpec((D, DO), lambda: (0, 0)),
            pl.BlockSpec((1, DO), lambda: (0, 0)),
        ],
        out_specs=[
            pl.BlockSpec((TOPS, 1), lambda: (0, 0)),
            pl.BlockSpec((TOPS, DO), lambda: (0, 0)),
            pl.BlockSpec((1, DO), lambda: (0, 0)),
        ],
        out_shape=[
            jax.ShapeDtypeStruct((TOPS, 1), jnp.float32),
            jax.ShapeDtypeStruct((TOPS, DO), jnp.float32),
            jax.ShapeDtypeStruct((1, DO), jnp.float32),
        ],
    )(scene_x, sub_embs, w, b2)


def kernel(scene_x, graph_x, W, b):
    t = _k1(scene_x, graph_x)
    md, y = _k1b(t, scene_x)
    cos = _k2(y, graph_x)
    cidx = _k3(cos)[:, 0]
    sub_embs = _sc_gather(cidx, graph_x)
    tmv, tsub, semb = _k4(scene_x, sub_embs, W, jnp.reshape(b, (1, DO)))
    return tmv[:, 0], tsub, semb[0], md[:, 0]
